# baseline (speedup 1.0000x reference)
import functools

import jax
import jax.numpy as jnp
from jax import lax
from jax.experimental import pallas as pl
from jax.experimental.pallas import tpu as pltpu
from jax.experimental.pallas import tpu_sc as plsc

B = 16384
D = 32
G = 4
CHUNK = 128


def kernel(user_id, item_id, context_id, user_table, item_table, context_table):
    info = plsc.get_sparse_core_info()
    nc, ns = info.num_cores, info.num_subcores
    nw = nc * ns
    b_per_w = B // nw
    n_chunks = b_per_w // CHUNK

    ut4 = user_table.reshape(-1, G * D)
    it4 = item_table.reshape(-1, G * D)
    ct4 = context_table.reshape(-1, G * D)

    mesh = plsc.VectorSubcoreMesh(core_axis_name="c", subcore_axis_name="s")

    @functools.partial(
        pl.kernel,
        mesh=mesh,
        out_type=jax.ShapeDtypeStruct((B * D,), jnp.float32),
        compiler_params=pltpu.CompilerParams(needs_layout_passes=False),
        scratch_types=[
            pltpu.VMEM((b_per_w,), jnp.int32),
            pltpu.VMEM((b_per_w,), jnp.int32),
            pltpu.VMEM((b_per_w,), jnp.int32),
            pltpu.VMEM((b_per_w,), jnp.int32),
            pltpu.VMEM((b_per_w,), jnp.int32),
            pltpu.VMEM((b_per_w,), jnp.int32),
            pltpu.VMEM((CHUNK, G * D), jnp.float32),
            pltpu.VMEM((CHUNK, G * D), jnp.float32),
            pltpu.VMEM((CHUNK, G * D), jnp.float32),
            pltpu.VMEM((b_per_w * D,), jnp.float32),
            pltpu.SemaphoreType.DMA,
            pltpu.SemaphoreType.DMA,
            pltpu.SemaphoreType.DMA,
        ],
    )
    def sc_kernel(uid_hbm, iid_hbm, cid_hbm, ut_hbm, it_hbm, ct_hbm, out_hbm,
                  uidx_v, iidx_v, cidx_v, ug_v, ig_v, cg_v,
                  ubuf, ibuf, cbuf, out_v, sem_u, sem_i, sem_c):
        wid = lax.axis_index("s") * nc + lax.axis_index("c")
        base = wid * b_per_w
        pltpu.sync_copy(uid_hbm.at[pl.ds(base, b_per_w)], uidx_v)
        pltpu.sync_copy(iid_hbm.at[pl.ds(base, b_per_w)], iidx_v)
        pltpu.sync_copy(cid_hbm.at[pl.ds(base, b_per_w)], cidx_v)

        def idx_body(k, carry):
            s = pl.ds(k * 16, 16)
            ug_v[s] = lax.shift_right_logical(uidx_v[s], 2)
            ig_v[s] = lax.shift_right_logical(iidx_v[s], 2)
            cg_v[s] = lax.shift_right_logical(cidx_v[s], 2)
            return carry

        lax.fori_loop(0, b_per_w // 16, idx_body, 0)

        third = jnp.float32(1.0 / 3.0)
        lane = lax.iota(jnp.int32, 16)

        for c in range(n_chunks):
            cs = pl.ds(c * CHUNK, CHUNK)
            cu = pltpu.async_copy(ut_hbm.at[ug_v.at[cs]], ubuf, sem_u)
            ci = pltpu.async_copy(it_hbm.at[ig_v.at[cs]], ibuf, sem_i)
            cc = pltpu.async_copy(ct_hbm.at[cg_v.at[cs]], cbuf, sem_c)
            cu.wait()
            ci.wait()
            cc.wait()

            def sel_body(k, carry):
                r0 = c * CHUNK + k * 16
                rows = k * 16 + lane
                s16 = pl.ds(r0, 16)
                su = (uidx_v[s16] & 3) * D
                si = (iidx_v[s16] & 3) * D
                sc = (cidx_v[s16] & 3) * D
                orow = (r0 + lane) * D
                for col in range(D):
                    vu = plsc.load_gather(ubuf, [rows, su + col])
                    vi = plsc.load_gather(ibuf, [rows, si + col])
                    vc = plsc.load_gather(cbuf, [rows, sc + col])
                    plsc.store_scatter(out_v, [orow + col],
                                       (vu + vi + vc) * third)
                return carry

            lax.fori_loop(0, CHUNK // 16, sel_body, 0)

        pltpu.sync_copy(out_v, out_hbm.at[pl.ds(base * D, b_per_w * D)])

    out_flat = sc_kernel(user_id, item_id, context_id, ut4, it4, ct4)
    return out_flat.reshape(B, D)


# speedup vs baseline: 1.0705x; 1.0705x over previous
import functools

import jax
import jax.numpy as jnp
from jax import lax
from jax.experimental import pallas as pl
from jax.experimental.pallas import tpu as pltpu
from jax.experimental.pallas import tpu_sc as plsc

B = 16384
D = 32
W = 128
CHUNK = 128


def kernel(user_id, item_id, context_id, user_table, item_table, context_table):
    info = plsc.get_sparse_core_info()
    nc, ns = info.num_cores, info.num_subcores
    nw = nc * ns
    b_per_w = B // nw
    n_chunks = b_per_w // CHUNK

    pad = ((0, 0), (0, W - D))
    utp = jnp.pad(user_table, pad)
    itp = jnp.pad(item_table, pad)
    ctp = jnp.pad(context_table, pad)

    mesh = plsc.VectorSubcoreMesh(core_axis_name="c", subcore_axis_name="s")

    @functools.partial(
        pl.kernel,
        mesh=mesh,
        out_type=jax.ShapeDtypeStruct((B * D,), jnp.float32),
        scratch_types=[
            pltpu.VMEM((b_per_w,), jnp.int32),
            pltpu.VMEM((b_per_w,), jnp.int32),
            pltpu.VMEM((b_per_w,), jnp.int32),
            pltpu.VMEM((CHUNK, W), jnp.float32),
            pltpu.VMEM((CHUNK, W), jnp.float32),
            pltpu.VMEM((CHUNK, W), jnp.float32),
            pltpu.VMEM((b_per_w * D,), jnp.float32),
            pltpu.SemaphoreType.DMA,
            pltpu.SemaphoreType.DMA,
            pltpu.SemaphoreType.DMA,
        ],
    )
    def sc_kernel(uid_hbm, iid_hbm, cid_hbm, ut_hbm, it_hbm, ct_hbm, out_hbm,
                  uidx_v, iidx_v, cidx_v, ubuf, ibuf, cbuf, out_v,
                  sem_u, sem_i, sem_c):
        wid = lax.axis_index("s") * nc + lax.axis_index("c")
        base = wid * b_per_w
        pltpu.sync_copy(uid_hbm.at[pl.ds(base, b_per_w)], uidx_v)
        pltpu.sync_copy(iid_hbm.at[pl.ds(base, b_per_w)], iidx_v)
        pltpu.sync_copy(cid_hbm.at[pl.ds(base, b_per_w)], cidx_v)

        third = jnp.float32(1.0 / 3.0)

        for c in range(n_chunks):
            cs = pl.ds(c * CHUNK, CHUNK)
            cu = pltpu.async_copy(ut_hbm.at[uidx_v.at[cs]], ubuf, sem_u)
            ci = pltpu.async_copy(it_hbm.at[iidx_v.at[cs]], ibuf, sem_i)
            cc = pltpu.async_copy(ct_hbm.at[cidx_v.at[cs]], cbuf, sem_c)
            cu.wait()
            ci.wait()
            cc.wait()

            def avg_body(r, carry):
                for col in range(0, D, 16):
                    s = pl.ds(col, 16)
                    out_v[pl.ds((c * CHUNK + r) * D + col, 16)] = (
                        ubuf[r, s] + ibuf[r, s] + cbuf[r, s]) * third
                return carry

            lax.fori_loop(0, CHUNK, avg_body, 0)

        pltpu.sync_copy(out_v, out_hbm.at[pl.ds(base * D, b_per_w * D)])

    out_flat = sc_kernel(user_id, item_id, context_id, utp, itp, ctp)
    return out_flat.reshape(B, D)


# speedup vs baseline: 1.6365x; 1.5288x over previous
import functools

import jax
import jax.numpy as jnp
from jax import lax
from jax.experimental import pallas as pl
from jax.experimental.pallas import tpu as pltpu
from jax.experimental.pallas import tpu_sc as plsc

B = 16384
D = 32
CHUNK = 128


def kernel(user_id, item_id, context_id, user_table, item_table, context_table):
    info = plsc.get_sparse_core_info()
    nc, ns = info.num_cores, info.num_subcores
    nw = nc * ns
    b_per_w = B // nw
    n_chunks = b_per_w // CHUNK

    mesh = plsc.VectorSubcoreMesh(core_axis_name="c", subcore_axis_name="s")

    @functools.partial(
        pl.kernel,
        mesh=mesh,
        out_type=jax.ShapeDtypeStruct((B * D,), jnp.float32),
        scratch_types=[
            pltpu.SMEM((b_per_w,), jnp.int32),
            pltpu.SMEM((b_per_w,), jnp.int32),
            pltpu.SMEM((b_per_w,), jnp.int32),
            pltpu.MemorySpace.VMEM_SHARED((ns, 3 * b_per_w), jnp.int32),
            pltpu.VMEM((CHUNK, D), jnp.float32),
            pltpu.VMEM((CHUNK, D), jnp.float32),
            pltpu.VMEM((CHUNK, D), jnp.float32),
            pltpu.VMEM((b_per_w * D,), jnp.float32),
            pltpu.SemaphoreType.DMA,
            pltpu.SemaphoreType.DMA,
            pltpu.SemaphoreType.DMA,
        ],
    )
    def sc_kernel(uid_hbm, iid_hbm, cid_hbm, ut_hbm, it_hbm, ct_hbm, out_hbm,
                  uid_s, iid_s, cid_s, ids_shr,
                  ubuf, ibuf, cbuf, out_v, sem_u, sem_i, sem_c):
        wid = lax.axis_index("s") * nc + lax.axis_index("c")
        sid = lax.axis_index("s")
        base = wid * b_per_w
        pltpu.sync_copy(uid_hbm.at[pl.ds(base, b_per_w)],
                        ids_shr.at[sid, pl.ds(0, b_per_w)])
        pltpu.sync_copy(iid_hbm.at[pl.ds(base, b_per_w)],
                        ids_shr.at[sid, pl.ds(b_per_w, b_per_w)])
        pltpu.sync_copy(cid_hbm.at[pl.ds(base, b_per_w)],
                        ids_shr.at[sid, pl.ds(2 * b_per_w, b_per_w)])
        pltpu.sync_copy(ids_shr.at[sid, pl.ds(0, b_per_w)], uid_s)
        pltpu.sync_copy(ids_shr.at[sid, pl.ds(b_per_w, b_per_w)], iid_s)
        pltpu.sync_copy(ids_shr.at[sid, pl.ds(2 * b_per_w, b_per_w)], cid_s)

        third = jnp.float32(1.0 / 3.0)

        for c in range(n_chunks):
            def fire_body(r, carry):
                dst = pl.ds(r, 1)
                pltpu.async_copy(ut_hbm.at[pl.ds(uid_s[c * CHUNK + r], 1)],
                                 ubuf.at[dst], sem_u)
                pltpu.async_copy(it_hbm.at[pl.ds(iid_s[c * CHUNK + r], 1)],
                                 ibuf.at[dst], sem_i)
                pltpu.async_copy(ct_hbm.at[pl.ds(cid_s[c * CHUNK + r], 1)],
                                 cbuf.at[dst], sem_c)
                return carry

            lax.fori_loop(0, CHUNK, fire_body, 0)
            pltpu.make_async_copy(ut_hbm.at[pl.ds(0, CHUNK)], ubuf,
                                  sem_u).wait()
            pltpu.make_async_copy(it_hbm.at[pl.ds(0, CHUNK)], ibuf,
                                  sem_i).wait()
            pltpu.make_async_copy(ct_hbm.at[pl.ds(0, CHUNK)], cbuf,
                                  sem_c).wait()

            def avg_body(r, carry):
                for col in range(0, D, 16):
                    s = pl.ds(col, 16)
                    out_v[pl.ds((c * CHUNK + r) * D + col, 16)] = (
                        ubuf[r, s] + ibuf[r, s] + cbuf[r, s]) * third
                return carry

            lax.fori_loop(0, CHUNK, avg_body, 0)

        pltpu.sync_copy(out_v, out_hbm.at[pl.ds(base * D, b_per_w * D)])

    out_flat = sc_kernel(user_id, item_id, context_id,
                         user_table, item_table, context_table)
    return out_flat.reshape(B, D)
